# baseline (speedup 1.0000x reference)
import functools

import jax
import jax.numpy as jnp
from jax import lax
from jax.experimental import pallas as pl
from jax.experimental.pallas import tpu as pltpu
from jax.experimental.pallas import tpu_sc as plsc

NU = 50000
NI = 50000
N = NU + NI
D = 64
QW = 16
NQ = D // QW
E = 800000
LANE = 128
ROWS = 6272
EPAD = ROWS * LANE - E
RPT = ROWS // 16
BLKS = RPT // 8
IB = 8
NBLK = RPT // IB
ACC_ROWS = 50048
TRASH = 50000
SLICE = 3128
WOUT_LAST = NU - 15 * SLICE
ZROWS = 782
ACC1 = 50176
CPAD = 102400
ZB1 = 3136

_f32 = jnp.float32


@functools.lru_cache(maxsize=None)
def _sc_kernels():
    mesh = plsc.VectorSubcoreMesh(
        core_axis_name="c", subcore_axis_name="s", num_cores=2, num_subcores=16
    )

    @functools.partial(
        pl.kernel,
        out_type=jax.ShapeDtypeStruct((CPAD,), _f32),
        mesh=mesh,
        compiler_params=pltpu.CompilerParams(use_tc_tiling_on_sc=False),
        scratch_types=[
            pltpu.VMEM((8, LANE), jnp.int32),
            pltpu.VMEM((LANE,), _f32),
            pltpu.VMEM((ZB1,), _f32),
            pltpu.VMEM_SHARED((ACC1,), _f32),
            pltpu.SemaphoreType.DMA,
        ],
    )
    def counts_k(d0, d1, cnt, didx, ones_v, zb1, acc1, ssem):
        c = lax.axis_index("c")
        s = lax.axis_index("s")

        @pl.loop(0, 8)
        def _f1(i):
            ones_v[pl.ds(i * 16, 16)] = jnp.ones((16,), _f32)

        @pl.loop(0, ZB1 // 16)
        def _fz(i):
            zb1[pl.ds(i * 16, 16)] = jnp.zeros((16,), _f32)

        pltpu.sync_copy(zb1, acc1.at[pl.ds(s * ZB1, ZB1)])
        plsc.subcore_barrier()

        def run(dstr, base):
            @pl.loop(0, BLKS)
            def _blk(b):
                rowbase = s * RPT + b * 8
                pltpu.sync_copy(dstr.at[pl.ds(rowbase, 8)], didx)
                scs = [
                    pltpu.async_copy(ones_v, acc1.at[didx.at[j]], ssem, add=True)
                    for j in range(8)
                ]
                for dsc in scs:
                    dsc.wait()

            plsc.subcore_barrier()

            @pl.when(s < 15)
            def _():
                pltpu.sync_copy(acc1.at[pl.ds(s * SLICE, SLICE)],
                                zb1.at[pl.ds(0, SLICE)])
                pltpu.sync_copy(zb1.at[pl.ds(0, SLICE)],
                                cnt.at[pl.ds(base + s * SLICE, SLICE)])

            @pl.when(s == 15)
            def _():
                pltpu.sync_copy(acc1.at[pl.ds(15 * SLICE, WOUT_LAST)],
                                zb1.at[pl.ds(0, WOUT_LAST)])
                pltpu.sync_copy(zb1.at[pl.ds(0, WOUT_LAST)],
                                cnt.at[pl.ds(base + 15 * SLICE, WOUT_LAST)])

        @pl.when(c == 0)
        def _():
            run(d0, 0)

        @pl.when(c == 1)
        def _():
            run(d1, NU)

    @functools.partial(
        pl.kernel,
        out_type=jax.ShapeDtypeStruct((N, D), _f32),
        mesh=mesh,
        compiler_params=pltpu.CompilerParams(use_tc_tiling_on_sc=False),
        scratch_types=[
            pltpu.VMEM((2, IB, LANE), jnp.int32),
            pltpu.VMEM((2, IB, LANE), jnp.int32),
            pltpu.VMEM((2, IB * LANE, QW), _f32),
            pltpu.VMEM((ZROWS, QW), _f32),
            pltpu.VMEM_SHARED((ACC_ROWS, QW), _f32),
            pltpu.SemaphoreType.DMA,
            pltpu.SemaphoreType.DMA,
            pltpu.SemaphoreType.DMA,
        ],
    )
    def layer_k(y4, s0, d0, s1, d1, z,
                sbuf, dbuf, rows, zbuf, acc, isem, gsem, ssem):
        c = lax.axis_index("c")
        s = lax.axis_index("s")

        @pl.loop(0, ZROWS)
        def _fz(i):
            zbuf[i] = jnp.zeros((QW,), _f32)

        def run(srcr, dstr, zbase):
            def idx_start(blk, P):
                rb = s * RPT + blk * IB
                pltpu.async_copy(srcr.at[pl.ds(rb, IB)], sbuf.at[P], isem)
                pltpu.async_copy(dstr.at[pl.ds(rb, IB)], dbuf.at[P], isem)

            def idx_wait(blk, P):
                rb = s * RPT + blk * IB
                pltpu.make_async_copy(
                    srcr.at[pl.ds(rb, IB)], sbuf.at[P], isem
                ).wait()
                pltpu.make_async_copy(
                    dstr.at[pl.ds(rb, IB)], dbuf.at[P], isem
                ).wait()

            for p in range(NQ):
                ytab = y4.at[pl.ds(p, 4 * N - p)]

                def g_start(P):
                    for j in range(IB):
                        pltpu.async_copy(
                            ytab.at[sbuf.at[P, j]],
                            rows.at[P, pl.ds(j * LANE, LANE)],
                            gsem,
                        )

                def g_wait(P):
                    for j in range(IB):
                        pltpu.make_async_copy(
                            ytab.at[sbuf.at[P, j]],
                            rows.at[P, pl.ds(j * LANE, LANE)],
                            gsem,
                        ).wait()

                def s_start(P):
                    for j in range(IB):
                        pltpu.async_copy(
                            rows.at[P, pl.ds(j * LANE, LANE)],
                            acc.at[dbuf.at[P, j]],
                            ssem,
                            add=True,
                        )

                def s_wait(P):
                    for j in range(IB):
                        pltpu.make_async_copy(
                            rows.at[P, pl.ds(j * LANE, LANE)],
                            acc.at[dbuf.at[P, j]],
                            ssem,
                        ).wait()

                idx_start(0, 0)
                idx_start(1, 1)
                zd = [
                    pltpu.async_copy(
                        zbuf, acc.at[pl.ds(s * SLICE + k * ZROWS, ZROWS)],
                        ssem,
                    )
                    for k in range(4)
                ]
                idx_wait(0, 0)
                g_start(0)
                for dsc in zd:
                    dsc.wait()
                plsc.subcore_barrier()

                @pl.loop(0, NBLK - 2, step=2)
                def _it(b):
                    g_wait(0)
                    s_start(0)
                    idx_wait(b + 1, 1)
                    g_start(1)
                    s_wait(0)
                    idx_start(b + 2, 0)

                    g_wait(1)
                    s_start(1)
                    idx_wait(b + 2, 0)
                    g_start(0)
                    s_wait(1)

                    @pl.when(b + 3 <= NBLK - 1)
                    def _():
                        idx_start(b + 3, 1)

                g_wait(0)
                s_start(0)
                s_wait(0)

                plsc.subcore_barrier()

                @pl.when(s < 15)
                def _():
                    pltpu.sync_copy(
                        acc.at[pl.ds(s * SLICE, SLICE)],
                        z.at[pl.ds(zbase + s * SLICE, SLICE),
                             pl.ds(p * QW, QW)],
                    )

                @pl.when(s == 15)
                def _():
                    pltpu.sync_copy(
                        acc.at[pl.ds(15 * SLICE, WOUT_LAST)],
                        z.at[pl.ds(zbase + 15 * SLICE, WOUT_LAST),
                             pl.ds(p * QW, QW)],
                    )

        @pl.when(c == 0)
        def _():
            run(s0, d0, 0)

        @pl.when(c == 1)
        def _():
            run(s1, d1, NU)

    return counts_k, layer_k


_W2R = N // 2
_RB = 2000
_GRID = _W2R // _RB


def _sb(c_ref):
    cblk = c_ref[...]
    ad = jnp.where(cblk == 0.0, jnp.float32(1e-6), cblk)
    return lax.rsqrt(ad)


def _prescale_body(w_ref, c_ref, y_ref):
    y_ref[...] = w_ref[...] * _sb(c_ref)


_prescale = pl.pallas_call(
    _prescale_body,
    grid=(_GRID,),
    in_specs=[
        pl.BlockSpec((_RB, 128), lambda i: (i, 0)),
        pl.BlockSpec((_RB, 128), lambda i: (i, 0)),
    ],
    out_specs=pl.BlockSpec((_RB, 128), lambda i: (i, 0)),
    out_shape=jax.ShapeDtypeStruct((_W2R, 128), _f32),
)


def _mid_body(z_ref, c_ref, y_ref):
    sb = _sb(c_ref)
    y_ref[...] = z_ref[...] * (sb * sb)


_mid = pl.pallas_call(
    _mid_body,
    grid=(_GRID,),
    in_specs=[
        pl.BlockSpec((_RB, 128), lambda i: (i, 0)),
        pl.BlockSpec((_RB, 128), lambda i: (i, 0)),
    ],
    out_specs=pl.BlockSpec((_RB, 128), lambda i: (i, 0)),
    out_shape=jax.ShapeDtypeStruct((_W2R, 128), _f32),
)


def _final_body(w_ref, z1_ref, z2_ref, c_ref, o_ref):
    o_ref[...] = (w_ref[...] + (z1_ref[...] + z2_ref[...]) * _sb(c_ref)) / 3.0


_final = pl.pallas_call(
    _final_body,
    grid=(_GRID,),
    in_specs=[
        pl.BlockSpec((_RB, 128), lambda i: (i, 0)),
        pl.BlockSpec((_RB, 128), lambda i: (i, 0)),
        pl.BlockSpec((_RB, 128), lambda i: (i, 0)),
        pl.BlockSpec((_RB, 128), lambda i: (i, 0)),
    ],
    out_specs=pl.BlockSpec((_RB, 128), lambda i: (i, 0)),
    out_shape=jax.ShapeDtypeStruct((_W2R, 128), _f32),
)


def kernel(weight, train_user, train_item):
    counts_k, layer_k = _sc_kernels()

    ti = train_item + NU
    pad0 = jnp.zeros((EPAD,), jnp.int32)
    padt = jnp.full((EPAD,), TRASH, jnp.int32)
    src0 = jnp.concatenate([ti * 4, pad0]).reshape(ROWS, LANE)
    dst0 = jnp.concatenate([train_user, padt]).reshape(ROWS, LANE)
    src1 = jnp.concatenate([train_user * 4, pad0]).reshape(ROWS, LANE)
    dst1 = jnp.concatenate([train_item, padt]).reshape(ROWS, LANE)

    cnt = counts_k(dst0, dst1)
    srep = jnp.broadcast_to(cnt[:N, None], (N, D)).reshape(_W2R, 128)
    w2 = weight.reshape(_W2R, 128)

    y0 = _prescale(w2, srep)
    z1 = layer_k(y0.reshape(4 * N, QW), src0, dst0, src1, dst1)
    z1_2 = z1.reshape(_W2R, 128)
    y1 = _mid(z1_2, srep)
    z2 = layer_k(y1.reshape(4 * N, QW), src0, dst0, src1, dst1)
    out = _final(w2, z1_2, z2.reshape(_W2R, 128), srep)
    return out.reshape(N, D)[:NU], out.reshape(N, D)[NU:]


# speedup vs baseline: 1.0039x; 1.0039x over previous
import functools

import jax
import jax.numpy as jnp
from jax import lax
from jax.experimental import pallas as pl
from jax.experimental.pallas import tpu as pltpu
from jax.experimental.pallas import tpu_sc as plsc

NU = 50000
NI = 50000
N = NU + NI
D = 64
QW = 16
NQ = D // QW
E = 800000
LANE = 128
ROWS = 6272
EPAD = ROWS * LANE - E
RPT = ROWS // 16
BLKS = RPT // 8
IB = 8
NBLK = RPT // IB
ACC_ROWS = 50048
TRASH = 50000
SLICE = 3128
WOUT_LAST = NU - 15 * SLICE
ZROWS = 782
ACC1 = 50176
CPAD = 102400
ZB1 = 3136

_f32 = jnp.float32


@functools.lru_cache(maxsize=None)
def _sc_kernels():
    mesh = plsc.VectorSubcoreMesh(
        core_axis_name="c", subcore_axis_name="s", num_cores=2, num_subcores=16
    )

    @functools.partial(
        pl.kernel,
        out_type=jax.ShapeDtypeStruct((CPAD,), _f32),
        mesh=mesh,
        compiler_params=pltpu.CompilerParams(use_tc_tiling_on_sc=False),
        scratch_types=[
            pltpu.VMEM((8, LANE), jnp.int32),
            pltpu.VMEM((LANE,), _f32),
            pltpu.VMEM((ZB1,), _f32),
            pltpu.VMEM_SHARED((ACC1,), _f32),
            pltpu.SemaphoreType.DMA,
        ],
    )
    def counts_k(d0, d1, cnt, didx, ones_v, zb1, acc1, ssem):
        c = lax.axis_index("c")
        s = lax.axis_index("s")

        @pl.loop(0, 8)
        def _f1(i):
            ones_v[pl.ds(i * 16, 16)] = jnp.ones((16,), _f32)

        @pl.loop(0, ZB1 // 16)
        def _fz(i):
            zb1[pl.ds(i * 16, 16)] = jnp.zeros((16,), _f32)

        pltpu.sync_copy(zb1, acc1.at[pl.ds(s * ZB1, ZB1)])
        plsc.subcore_barrier()

        def run(dstr, base):
            @pl.loop(0, BLKS)
            def _blk(b):
                rowbase = s * RPT + b * 8
                pltpu.sync_copy(dstr.at[pl.ds(rowbase, 8)], didx)
                scs = [
                    pltpu.async_copy(ones_v, acc1.at[didx.at[j]], ssem, add=True)
                    for j in range(8)
                ]
                for dsc in scs:
                    dsc.wait()

            plsc.subcore_barrier()

            @pl.when(s < 15)
            def _():
                pltpu.sync_copy(acc1.at[pl.ds(s * SLICE, SLICE)],
                                zb1.at[pl.ds(0, SLICE)])
                pltpu.sync_copy(zb1.at[pl.ds(0, SLICE)],
                                cnt.at[pl.ds(base + s * SLICE, SLICE)])

            @pl.when(s == 15)
            def _():
                pltpu.sync_copy(acc1.at[pl.ds(15 * SLICE, WOUT_LAST)],
                                zb1.at[pl.ds(0, WOUT_LAST)])
                pltpu.sync_copy(zb1.at[pl.ds(0, WOUT_LAST)],
                                cnt.at[pl.ds(base + 15 * SLICE, WOUT_LAST)])

        @pl.when(c == 0)
        def _():
            run(d0, 0)

        @pl.when(c == 1)
        def _():
            run(d1, NU)

    @functools.partial(
        pl.kernel,
        out_type=jax.ShapeDtypeStruct((N, D), _f32),
        mesh=mesh,
        compiler_params=pltpu.CompilerParams(use_tc_tiling_on_sc=False),
        scratch_types=[
            pltpu.VMEM((2, IB, LANE), jnp.int32),
            pltpu.VMEM((2, IB, LANE), jnp.int32),
            pltpu.VMEM((2, IB * LANE, QW), _f32),
            pltpu.VMEM((ZROWS, QW), _f32),
            pltpu.VMEM_SHARED((ACC_ROWS, QW), _f32),
            pltpu.SemaphoreType.DMA,
            pltpu.SemaphoreType.DMA,
            pltpu.SemaphoreType.DMA,
        ],
    )
    def layer_k(y4, s0, d0, s1, d1, z,
                sbuf, dbuf, rows, zbuf, acc, isem, gsem, ssem):
        c = lax.axis_index("c")
        s = lax.axis_index("s")

        @pl.loop(0, ZROWS)
        def _fz(i):
            zbuf[i] = jnp.zeros((QW,), _f32)

        def run(srcr, dstr, zbase):
            def idx_start(blk, P):
                rb = s * RPT + blk * IB
                pltpu.async_copy(srcr.at[pl.ds(rb, IB)], sbuf.at[P], isem)
                pltpu.async_copy(dstr.at[pl.ds(rb, IB)], dbuf.at[P], isem)

            def idx_wait(blk, P):
                rb = s * RPT + blk * IB
                pltpu.make_async_copy(
                    srcr.at[pl.ds(rb, IB)], sbuf.at[P], isem
                ).wait()
                pltpu.make_async_copy(
                    dstr.at[pl.ds(rb, IB)], dbuf.at[P], isem
                ).wait()

            for p in range(NQ):
                ytab = y4.at[pl.ds(p, 4 * N - p)]

                def g_start(P):
                    for j in range(IB):
                        pltpu.async_copy(
                            ytab.at[sbuf.at[P, j]],
                            rows.at[P, pl.ds(j * LANE, LANE)],
                            gsem,
                        )

                def g_wait(P):
                    pltpu.make_async_copy(
                        y4.at[pl.ds(0, IB * LANE)], rows.at[P], gsem
                    ).wait()

                def s_start(P):
                    for j in range(IB):
                        pltpu.async_copy(
                            rows.at[P, pl.ds(j * LANE, LANE)],
                            acc.at[dbuf.at[P, j]],
                            ssem,
                            add=True,
                        )

                def s_wait(P):
                    pltpu.make_async_copy(
                        rows.at[P], acc.at[pl.ds(0, IB * LANE)], ssem
                    ).wait()

                idx_start(0, 0)
                idx_start(1, 1)
                zd = [
                    pltpu.async_copy(
                        zbuf, acc.at[pl.ds(s * SLICE + k * ZROWS, ZROWS)],
                        ssem,
                    )
                    for k in range(4)
                ]
                idx_wait(0, 0)
                g_start(0)
                for dsc in zd:
                    dsc.wait()
                plsc.subcore_barrier()

                @pl.loop(0, NBLK - 2, step=2)
                def _it(b):
                    g_wait(0)
                    s_start(0)
                    idx_wait(b + 1, 1)
                    g_start(1)
                    s_wait(0)
                    idx_start(b + 2, 0)

                    g_wait(1)
                    s_start(1)
                    idx_wait(b + 2, 0)
                    g_start(0)
                    s_wait(1)

                    @pl.when(b + 3 <= NBLK - 1)
                    def _():
                        idx_start(b + 3, 1)

                g_wait(0)
                s_start(0)
                s_wait(0)

                plsc.subcore_barrier()

                @pl.when(s < 15)
                def _():
                    pltpu.sync_copy(
                        acc.at[pl.ds(s * SLICE, SLICE)],
                        z.at[pl.ds(zbase + s * SLICE, SLICE),
                             pl.ds(p * QW, QW)],
                    )

                @pl.when(s == 15)
                def _():
                    pltpu.sync_copy(
                        acc.at[pl.ds(15 * SLICE, WOUT_LAST)],
                        z.at[pl.ds(zbase + 15 * SLICE, WOUT_LAST),
                             pl.ds(p * QW, QW)],
                    )

        @pl.when(c == 0)
        def _():
            run(s0, d0, 0)

        @pl.when(c == 1)
        def _():
            run(s1, d1, NU)

    return counts_k, layer_k


_W2R = N // 2
_RB = 2000
_GRID = _W2R // _RB


def _sb(c_ref):
    cblk = c_ref[...]
    ad = jnp.where(cblk == 0.0, jnp.float32(1e-6), cblk)
    return lax.rsqrt(ad)


def _prescale_body(w_ref, c_ref, y_ref):
    y_ref[...] = w_ref[...] * _sb(c_ref)


_prescale = pl.pallas_call(
    _prescale_body,
    grid=(_GRID,),
    in_specs=[
        pl.BlockSpec((_RB, 128), lambda i: (i, 0)),
        pl.BlockSpec((_RB, 128), lambda i: (i, 0)),
    ],
    out_specs=pl.BlockSpec((_RB, 128), lambda i: (i, 0)),
    out_shape=jax.ShapeDtypeStruct((_W2R, 128), _f32),
)


def _mid_body(z_ref, c_ref, y_ref):
    sb = _sb(c_ref)
    y_ref[...] = z_ref[...] * (sb * sb)


_mid = pl.pallas_call(
    _mid_body,
    grid=(_GRID,),
    in_specs=[
        pl.BlockSpec((_RB, 128), lambda i: (i, 0)),
        pl.BlockSpec((_RB, 128), lambda i: (i, 0)),
    ],
    out_specs=pl.BlockSpec((_RB, 128), lambda i: (i, 0)),
    out_shape=jax.ShapeDtypeStruct((_W2R, 128), _f32),
)


def _final_body(w_ref, z1_ref, z2_ref, c_ref, o_ref):
    o_ref[...] = (w_ref[...] + (z1_ref[...] + z2_ref[...]) * _sb(c_ref)) / 3.0


_final = pl.pallas_call(
    _final_body,
    grid=(_GRID,),
    in_specs=[
        pl.BlockSpec((_RB, 128), lambda i: (i, 0)),
        pl.BlockSpec((_RB, 128), lambda i: (i, 0)),
        pl.BlockSpec((_RB, 128), lambda i: (i, 0)),
        pl.BlockSpec((_RB, 128), lambda i: (i, 0)),
    ],
    out_specs=pl.BlockSpec((_RB, 128), lambda i: (i, 0)),
    out_shape=jax.ShapeDtypeStruct((_W2R, 128), _f32),
)


def kernel(weight, train_user, train_item):
    counts_k, layer_k = _sc_kernels()

    ti = train_item + NU
    pad0 = jnp.zeros((EPAD,), jnp.int32)
    padt = jnp.full((EPAD,), TRASH, jnp.int32)
    src0 = jnp.concatenate([ti * 4, pad0]).reshape(ROWS, LANE)
    dst0 = jnp.concatenate([train_user, padt]).reshape(ROWS, LANE)
    src1 = jnp.concatenate([train_user * 4, pad0]).reshape(ROWS, LANE)
    dst1 = jnp.concatenate([train_item, padt]).reshape(ROWS, LANE)

    cnt = counts_k(dst0, dst1)
    srep = jnp.broadcast_to(cnt[:N, None], (N, D)).reshape(_W2R, 128)
    w2 = weight.reshape(_W2R, 128)

    y0 = _prescale(w2, srep)
    z1 = layer_k(y0.reshape(4 * N, QW), src0, dst0, src1, dst1)
    z1_2 = z1.reshape(_W2R, 128)
    y1 = _mid(z1_2, srep)
    z2 = layer_k(y1.reshape(4 * N, QW), src0, dst0, src1, dst1)
    out = _final(w2, z1_2, z2.reshape(_W2R, 128), srep)
    return out.reshape(N, D)[:NU], out.reshape(N, D)[NU:]
